# baseline (speedup 1.0000x reference)
import jax
import jax.numpy as jnp
from jax import lax
from jax.experimental import pallas as pl
from jax.experimental.pallas import tpu as pltpu
from jax.experimental.pallas import tpu_sc as plsc

_N = 16384
_D = 128
_P = 262144
_M1 = 1.0
_M2 = 0.5
_M3 = 0.5

_BR = 128
_NB = _N // _BR
_GROWS = (_NB + 1) * _BR


_AUG = 136


def _augment_body(e_ref, lhs_ref, rhs_ref):
    e = e_ref[...]
    n = jnp.sum(e * e, axis=1, keepdims=True)
    one = jnp.ones((_N, 1), jnp.float32)
    pad = jnp.zeros((_N, _AUG - _D - 2), jnp.float32)
    lhs_ref[...] = jnp.concatenate(
        [-2.0 * e, n, one, pad], axis=1).astype(jnp.bfloat16)
    rhs_ref[...] = jnp.concatenate(
        [e, one, n, pad], axis=1).astype(jnp.bfloat16)


def _augment(e):
    return pl.pallas_call(
        _augment_body,
        out_shape=[jax.ShapeDtypeStruct((_N, _AUG), jnp.bfloat16),
                   jax.ShapeDtypeStruct((_N, _AUG), jnp.bfloat16)],
    )(e)


def _gram_body(l1_ref, l2_ref, rhs_ref, g_ref, acc1_ref, acc2_ref):
    r = pl.program_id(0)
    acc1_ref[...] = lax.dot_general(
        l1_ref[...], rhs_ref[...], (((1,), (1,)), ((), ())),
        preferred_element_type=jnp.float32)
    acc2_ref[...] = lax.dot_general(
        l2_ref[...], rhs_ref[...], (((1,), (1,)), ((), ())),
        preferred_element_type=jnp.float32)

    def chunk1(cb, _):
        g_ref[pl.ds((cb - r) * _BR, _BR), :] = acc1_ref[:, pl.ds(cb * _D, _D)]
        return 0

    def chunk2(cb, _):
        g_ref[pl.ds((cb + 1) * _BR, _BR), :] = acc2_ref[:, pl.ds(cb * _D, _D)]
        return 0

    lax.fori_loop(r, _NB, chunk1, 0)
    lax.fori_loop(_NB - 1 - r, _NB, chunk2, 0)


def _gram(lhs, rhs):
    return pl.pallas_call(
        _gram_body,
        grid=(_NB // 2,),
        in_specs=[
            pl.BlockSpec((_BR, _AUG), lambda i: (i, 0)),
            pl.BlockSpec((_BR, _AUG), lambda i: (_NB - 1 - i, 0)),
            pl.BlockSpec((_N, _AUG), lambda i: (0, 0)),
        ],
        out_specs=[
            pl.BlockSpec((_GROWS, _D), lambda i: (i, 0)),
        ],
        out_shape=[
            jax.ShapeDtypeStruct((_NB // 2 * _GROWS, _D), jnp.float32),
        ],
        scratch_shapes=[pltpu.VMEM((_BR, _N), jnp.float32),
                        pltpu.VMEM((_BR, _N), jnp.float32)],
        compiler_params=pltpu.CompilerParams(
            dimension_semantics=("arbitrary",)),
    )(lhs, lhs, rhs)[0]


_NC = 2
_NS = 16
_NW = _NC * _NS
_S = _P // _NW
_GSUB = 128
_NSUB = _S // _GSUB


def _sc_body(g_hbm, pos_hbm, neg_hbm, opp_hbm, onp_hbm, trip_hbm,
             out_hbm, idx_v, gia_v, gib_v, gva_v, gvb_v, sem):
    wid = lax.axis_index("s") * _NC + lax.axis_index("c")
    base = wid * _S

    def gflat(a, b):
        x = jnp.minimum(a, b)
        y = jnp.maximum(a, b)
        rb = x >> 7
        cb = y >> 7
        lo = rb < (_NB // 2)
        r = jnp.where(lo, rb, _NB - 1 - rb)
        slot = jnp.where(lo, cb - rb, cb + 1)
        return (r * (_GROWS * _D) + (slot << 14)
                + ((x & 127) << 7) + (y & 127))

    def build_pair(j, _):
        a = idx_v[pl.ds(j * 16, 16)]
        b = idx_v[pl.ds(_S + j * 16, 16)]
        gia_v[j // 8, pl.ds((j % 8) * 16, 16)] = gflat(a, b)

        @pl.when(j % 8 == 7)
        def _():
            t = j // 8
            pltpu.async_copy(g_hbm.at[gia_v.at[t]], gva_v.at[t], sem)

        return 0

    def drain_a(t, _):
        pltpu.make_async_copy(g_hbm.at[gia_v.at[t]], gva_v.at[t], sem).wait()
        return 0

    def drain_b(t, _):
        pltpu.make_async_copy(g_hbm.at[gib_v.at[t]], gvb_v.at[t], sem).wait()
        return 0

    src_off = pl.multiple_of(base, 8)
    dst_base = wid * _NSUB

    for s, ref in enumerate((pos_hbm, neg_hbm, opp_hbm, onp_hbm)):
        pltpu.sync_copy(ref.at[pl.ds(src_off, _S)], idx_v.at[pl.ds(0, _S)])
        pltpu.sync_copy(ref.at[pl.ds(_P + src_off, _S)],
                        idx_v.at[pl.ds(_S, _S)])
        lax.fori_loop(0, _S // 16, build_pair, 0)
        lax.fori_loop(0, _NSUB, drain_a, 0)
        dst = pl.multiple_of(s * (_P // _GSUB) + dst_base, 8)
        pltpu.sync_copy(gva_v, out_hbm.at[pl.ds(dst, _NSUB), :])

    def build_trip(j, _):
        a = idx_v[pl.ds(j * 16, 16)]
        p = idx_v[pl.ds(_S + j * 16, 16)]
        n = idx_v[pl.ds(2 * _S + j * 16, 16)]
        gia_v[j // 8, pl.ds((j % 8) * 16, 16)] = gflat(a, p)
        gib_v[j // 8, pl.ds((j % 8) * 16, 16)] = gflat(a, n)

        @pl.when(j % 8 == 7)
        def _():
            t = j // 8
            pltpu.async_copy(g_hbm.at[gia_v.at[t]], gva_v.at[t], sem)
            pltpu.async_copy(g_hbm.at[gib_v.at[t]], gvb_v.at[t], sem)

        return 0

    pltpu.sync_copy(trip_hbm.at[pl.ds(src_off, _S)], idx_v.at[pl.ds(0, _S)])
    pltpu.sync_copy(trip_hbm.at[pl.ds(_P + src_off, _S)],
                    idx_v.at[pl.ds(_S, _S)])
    pltpu.sync_copy(trip_hbm.at[pl.ds(2 * _P + src_off, _S)],
                    idx_v.at[pl.ds(2 * _S, _S)])
    lax.fori_loop(0, _S // 16, build_trip, 0)
    lax.fori_loop(0, _NSUB, drain_a, 0)
    lax.fori_loop(0, _NSUB, drain_b, 0)
    dst = pl.multiple_of(4 * (_P // _GSUB) + dst_base, 8)
    pltpu.sync_copy(gva_v, out_hbm.at[pl.ds(dst, _NSUB), :])
    dst = pl.multiple_of(5 * (_P // _GSUB) + dst_base, 8)
    pltpu.sync_copy(gvb_v, out_hbm.at[pl.ds(dst, _NSUB), :])


def _sc_stage(g_flat, pos, neg, opp, onp, trip):
    mesh = plsc.VectorSubcoreMesh(core_axis_name="c", subcore_axis_name="s")
    return pl.kernel(
        _sc_body,
        out_type=jax.ShapeDtypeStruct((6 * _P // _GSUB, _GSUB), jnp.float32),
        mesh=mesh,
        scratch_types=[
            pltpu.VMEM((3 * _S,), jnp.int32),
            pltpu.VMEM((_NSUB, _GSUB), jnp.int32),
            pltpu.VMEM((_NSUB, _GSUB), jnp.int32),
            pltpu.VMEM((_NSUB, _GSUB), jnp.float32),
            pltpu.VMEM((_NSUB, _GSUB), jnp.float32),
            pltpu.SemaphoreType.DMA,
        ],
        compiler_params=pltpu.CompilerParams(needs_layout_passes=False),
    )(g_flat, pos, neg, opp, onp, trip)


_BC = 16384


def _reduce_body(d2_ref, tot_ref, cnt_ref):
    i = pl.program_id(0)
    d = jnp.sqrt(jnp.maximum(d2_ref[...], 0.0) + 1e-12)
    l1 = jnp.maximum(d[0:1, :] - d[1:2, :] + _M1, 0.0)
    l2 = jnp.maximum(_M2 - d[2:3, :], 0.0)
    l3 = jnp.maximum(_M3 - d[3:4, :], 0.0)
    l4 = jnp.maximum(d[4:5, :] - d[5:6, :] + _M2 + _M3, 0.0)
    s = jnp.sum(l1) + jnp.sum(l2) + jnp.sum(l3) + jnp.sum(l4)
    c = (jnp.sum((l1 > 0).astype(jnp.float32))
         + jnp.sum((l2 > 0).astype(jnp.float32))
         + jnp.sum((l3 > 0).astype(jnp.float32))
         + jnp.sum((l4 > 0).astype(jnp.float32)))

    @pl.when(i == 0)
    def _():
        tot_ref[...] = jnp.zeros((1, 1), jnp.float32)
        cnt_ref[...] = jnp.zeros((1, 1), jnp.float32)

    tot_ref[...] += jnp.reshape(s, (1, 1))
    cnt_ref[...] += jnp.reshape(c, (1, 1))


def _reduce(d2):
    return pl.pallas_call(
        _reduce_body,
        grid=(_P // _BC,),
        in_specs=[pl.BlockSpec((6, _BC), lambda i: (0, i))],
        out_specs=[pl.BlockSpec((1, 1), lambda i: (0, 0)),
                   pl.BlockSpec((1, 1), lambda i: (0, 0))],
        out_shape=[jax.ShapeDtypeStruct((1, 1), jnp.float32),
                   jax.ShapeDtypeStruct((1, 1), jnp.float32)],
        compiler_params=pltpu.CompilerParams(
            dimension_semantics=("arbitrary",)),
    )(d2)


def kernel(embeddings, labels, positive_pairs, negative_pairs,
           overlap_positive_pairs, overlap_negative_pairs, triplets):
    del labels
    lhs, rhs = _augment(embeddings)
    g = _gram(lhs, rhs)
    d2 = _sc_stage(
        g.reshape(-1),
        positive_pairs.T.reshape(-1),
        negative_pairs.T.reshape(-1),
        overlap_positive_pairs.T.reshape(-1),
        overlap_negative_pairs.T.reshape(-1),
        triplets.T.reshape(-1),
    )
    tot, cnt = _reduce(d2.reshape(6, _P))
    t = tot[0, 0]
    c = cnt[0, 0]
    return jnp.where(c > 0, t / jnp.maximum(c, 1.0), jnp.float32(0.0))


# speedup vs baseline: 1.4519x; 1.4519x over previous
import jax
import jax.numpy as jnp
from jax import lax
from jax.experimental import pallas as pl
from jax.experimental.pallas import tpu as pltpu
from jax.experimental.pallas import tpu_sc as plsc

_N = 16384
_D = 128
_P = 262144
_M1 = 1.0
_M2 = 0.5
_M3 = 0.5

_BR = 128
_NB = _N // _BR
_GROWS = (_NB + 1) * _BR


def _gram_body(e1_ref, e2_ref, e_all_ref, g_ref, acc1_ref, acc2_ref):
    r = pl.program_id(0)
    acc1_ref[...] = lax.dot_general(
        e1_ref[...], e_all_ref[...], (((1,), (1,)), ((), ())),
        preferred_element_type=jnp.float32)
    acc2_ref[...] = lax.dot_general(
        e2_ref[...], e_all_ref[...], (((1,), (1,)), ((), ())),
        preferred_element_type=jnp.float32)

    def chunk1(cb, _):
        g_ref[pl.ds((cb - r) * _BR, _BR), :] = acc1_ref[:, pl.ds(cb * _D, _D)]
        return 0

    def chunk2(cb, _):
        g_ref[pl.ds((cb + 1) * _BR, _BR), :] = acc2_ref[:, pl.ds(cb * _D, _D)]
        return 0

    lax.fori_loop(r, _NB, chunk1, 0)
    lax.fori_loop(_NB - 1 - r, _NB, chunk2, 0)


def _gram(e):
    return pl.pallas_call(
        _gram_body,
        grid=(_NB // 2,),
        in_specs=[
            pl.BlockSpec((_BR, _D), lambda i: (i, 0)),
            pl.BlockSpec((_BR, _D), lambda i: (_NB - 1 - i, 0)),
            pl.BlockSpec((_N, _D), lambda i: (0, 0)),
        ],
        out_specs=[
            pl.BlockSpec((_GROWS, _D), lambda i: (i, 0)),
        ],
        out_shape=[
            jax.ShapeDtypeStruct((_NB // 2 * _GROWS, _D), jnp.float32),
        ],
        scratch_shapes=[pltpu.VMEM((_BR, _N), jnp.float32),
                        pltpu.VMEM((_BR, _N), jnp.float32)],
        compiler_params=pltpu.CompilerParams(
            dimension_semantics=("arbitrary",)),
    )(e, e, e)[0]


def _norms_body(e_ref, nrm_ref):
    e = e_ref[...]
    nrm_ref[...] = jnp.sum(e * e, axis=1, keepdims=True)


def _norms(e):
    return pl.pallas_call(
        _norms_body,
        out_shape=jax.ShapeDtypeStruct((_N, 1), jnp.float32),
    )(e)


_NC = 2
_NS = 16
_NW = _NC * _NS
_S = _P // _NW
_GSUB = 128
_NSUB = _S // _GSUB


def _sc_body(g_hbm, pos_hbm, neg_hbm, opp_hbm, onp_hbm, trip_hbm, nrm_hbm,
             out_hbm, nrm_v, idx_v, gia_v, gib_v, gva_v, gvb_v,
             nsa_v, nsb_v, sem):
    wid = lax.axis_index("s") * _NC + lax.axis_index("c")
    base = wid * _S

    pltpu.sync_copy(nrm_hbm, nrm_v)

    def gflat(a, b):
        x = jnp.minimum(a, b)
        y = jnp.maximum(a, b)
        rb = x >> 7
        cb = y >> 7
        lo = rb < (_NB // 2)
        r = jnp.where(lo, rb, _NB - 1 - rb)
        slot = jnp.where(lo, cb - rb, cb + 1)
        return (r * (_GROWS * _D) + (slot << 14)
                + ((x & 127) << 7) + (y & 127))

    def build_pair(j, _):
        a = idx_v[pl.ds(j * 16, 16)]
        b = idx_v[pl.ds(_S + j * 16, 16)]
        gia_v[j // 8, pl.ds((j % 8) * 16, 16)] = gflat(a, b)
        nsa_v[j // 8, pl.ds((j % 8) * 16, 16)] = (
            plsc.load_gather(nrm_v, [a]) + plsc.load_gather(nrm_v, [b]))

        @pl.when(j % 8 == 7)
        def _():
            t = j // 8
            pltpu.async_copy(g_hbm.at[gia_v.at[t]], gva_v.at[t], sem)

        return 0

    def fin_a(j, _):
        nsa_v[j // 8, pl.ds((j % 8) * 16, 16)] = (
            nsa_v[j // 8, pl.ds((j % 8) * 16, 16)]
            - 2.0 * gva_v[j // 8, pl.ds((j % 8) * 16, 16)])
        return 0

    def fin_b(j, _):
        nsb_v[j // 8, pl.ds((j % 8) * 16, 16)] = (
            nsb_v[j // 8, pl.ds((j % 8) * 16, 16)]
            - 2.0 * gvb_v[j // 8, pl.ds((j % 8) * 16, 16)])
        return 0

    def drain_a(t, _):
        pltpu.make_async_copy(g_hbm.at[gia_v.at[t]], gva_v.at[t], sem).wait()
        return 0

    def drain_b(t, _):
        pltpu.make_async_copy(g_hbm.at[gib_v.at[t]], gvb_v.at[t], sem).wait()
        return 0

    src_off = pl.multiple_of(base, 8)
    dst_base = wid * _NSUB

    for s, ref in enumerate((pos_hbm, neg_hbm, opp_hbm, onp_hbm)):
        pltpu.sync_copy(ref.at[pl.ds(src_off, _S)], idx_v.at[pl.ds(0, _S)])
        pltpu.sync_copy(ref.at[pl.ds(_P + src_off, _S)],
                        idx_v.at[pl.ds(_S, _S)])
        lax.fori_loop(0, _S // 16, build_pair, 0)
        lax.fori_loop(0, _NSUB, drain_a, 0)
        lax.fori_loop(0, _S // 16, fin_a, 0)
        dst = pl.multiple_of(s * (_P // _GSUB) + dst_base, 8)
        pltpu.sync_copy(nsa_v, out_hbm.at[pl.ds(dst, _NSUB), :])

    def build_trip(j, _):
        a = idx_v[pl.ds(j * 16, 16)]
        p = idx_v[pl.ds(_S + j * 16, 16)]
        n = idx_v[pl.ds(2 * _S + j * 16, 16)]
        gia_v[j // 8, pl.ds((j % 8) * 16, 16)] = gflat(a, p)
        gib_v[j // 8, pl.ds((j % 8) * 16, 16)] = gflat(a, n)
        na = plsc.load_gather(nrm_v, [a])
        nsa_v[j // 8, pl.ds((j % 8) * 16, 16)] = (
            na + plsc.load_gather(nrm_v, [p]))
        nsb_v[j // 8, pl.ds((j % 8) * 16, 16)] = (
            na + plsc.load_gather(nrm_v, [n]))

        @pl.when(j % 8 == 7)
        def _():
            t = j // 8
            pltpu.async_copy(g_hbm.at[gia_v.at[t]], gva_v.at[t], sem)
            pltpu.async_copy(g_hbm.at[gib_v.at[t]], gvb_v.at[t], sem)

        return 0

    pltpu.sync_copy(trip_hbm.at[pl.ds(src_off, _S)], idx_v.at[pl.ds(0, _S)])
    pltpu.sync_copy(trip_hbm.at[pl.ds(_P + src_off, _S)],
                    idx_v.at[pl.ds(_S, _S)])
    pltpu.sync_copy(trip_hbm.at[pl.ds(2 * _P + src_off, _S)],
                    idx_v.at[pl.ds(2 * _S, _S)])
    lax.fori_loop(0, _S // 16, build_trip, 0)
    lax.fori_loop(0, _NSUB, drain_a, 0)
    lax.fori_loop(0, _NSUB, drain_b, 0)
    lax.fori_loop(0, _S // 16, fin_a, 0)
    lax.fori_loop(0, _S // 16, fin_b, 0)
    dst = pl.multiple_of(4 * (_P // _GSUB) + dst_base, 8)
    pltpu.sync_copy(nsa_v, out_hbm.at[pl.ds(dst, _NSUB), :])
    dst = pl.multiple_of(5 * (_P // _GSUB) + dst_base, 8)
    pltpu.sync_copy(nsb_v, out_hbm.at[pl.ds(dst, _NSUB), :])


def _sc_stage(g_flat, pos, neg, opp, onp, trip, nrm):
    mesh = plsc.VectorSubcoreMesh(core_axis_name="c", subcore_axis_name="s")
    return pl.kernel(
        _sc_body,
        out_type=jax.ShapeDtypeStruct((6 * _P // _GSUB, _GSUB), jnp.float32),
        mesh=mesh,
        scratch_types=[
            pltpu.VMEM((_N,), jnp.float32),
            pltpu.VMEM((3 * _S,), jnp.int32),
            pltpu.VMEM((_NSUB, _GSUB), jnp.int32),
            pltpu.VMEM((_NSUB, _GSUB), jnp.int32),
            pltpu.VMEM((_NSUB, _GSUB), jnp.float32),
            pltpu.VMEM((_NSUB, _GSUB), jnp.float32),
            pltpu.VMEM((_NSUB, _GSUB), jnp.float32),
            pltpu.VMEM((_NSUB, _GSUB), jnp.float32),
            pltpu.SemaphoreType.DMA,
        ],
        compiler_params=pltpu.CompilerParams(needs_layout_passes=False),
    )(g_flat, pos, neg, opp, onp, trip, nrm)


_BC = 16384


def _reduce_body(d2_ref, tot_ref, cnt_ref):
    i = pl.program_id(0)
    d = jnp.sqrt(jnp.maximum(d2_ref[...], 0.0) + 1e-12)
    l1 = jnp.maximum(d[0:1, :] - d[1:2, :] + _M1, 0.0)
    l2 = jnp.maximum(_M2 - d[2:3, :], 0.0)
    l3 = jnp.maximum(_M3 - d[3:4, :], 0.0)
    l4 = jnp.maximum(d[4:5, :] - d[5:6, :] + _M2 + _M3, 0.0)
    s = jnp.sum(l1) + jnp.sum(l2) + jnp.sum(l3) + jnp.sum(l4)
    c = (jnp.sum((l1 > 0).astype(jnp.float32))
         + jnp.sum((l2 > 0).astype(jnp.float32))
         + jnp.sum((l3 > 0).astype(jnp.float32))
         + jnp.sum((l4 > 0).astype(jnp.float32)))

    @pl.when(i == 0)
    def _():
        tot_ref[...] = jnp.zeros((1, 1), jnp.float32)
        cnt_ref[...] = jnp.zeros((1, 1), jnp.float32)

    tot_ref[...] += jnp.reshape(s, (1, 1))
    cnt_ref[...] += jnp.reshape(c, (1, 1))


def _reduce(d2):
    return pl.pallas_call(
        _reduce_body,
        grid=(_P // _BC,),
        in_specs=[pl.BlockSpec((6, _BC), lambda i: (0, i))],
        out_specs=[pl.BlockSpec((1, 1), lambda i: (0, 0)),
                   pl.BlockSpec((1, 1), lambda i: (0, 0))],
        out_shape=[jax.ShapeDtypeStruct((1, 1), jnp.float32),
                   jax.ShapeDtypeStruct((1, 1), jnp.float32)],
        compiler_params=pltpu.CompilerParams(
            dimension_semantics=("arbitrary",)),
    )(d2)


def kernel(embeddings, labels, positive_pairs, negative_pairs,
           overlap_positive_pairs, overlap_negative_pairs, triplets):
    del labels
    g = _gram(embeddings)
    nrm = _norms(embeddings)
    d2 = _sc_stage(
        g.reshape(-1),
        positive_pairs.T.reshape(-1),
        negative_pairs.T.reshape(-1),
        overlap_positive_pairs.T.reshape(-1),
        overlap_negative_pairs.T.reshape(-1),
        triplets.T.reshape(-1),
        nrm.reshape(-1),
    )
    tot, cnt = _reduce(d2.reshape(6, _P))
    t = tot[0, 0]
    c = cnt[0, 0]
    return jnp.where(c > 0, t / jnp.maximum(c, 1.0), jnp.float32(0.0))
